# baseline (speedup 1.0000x reference)
import functools

import jax
import jax.numpy as jnp
from jax import lax
from jax.experimental import pallas as pl
from jax.experimental.pallas import tpu as pltpu
from jax.experimental.pallas import tpu_sc as plsc

D_MODEL = 1024
NUM_WAVES = 3
N_TOKENS = 4 * 8192
NUM_CORES = 2
NUM_SUBCORES = 16
NUM_WORKERS = NUM_CORES * NUM_SUBCORES
TOK_PER_W = N_TOKENS // NUM_WORKERS
G = 2
D_G = G * D_MODEL
PAIRS_PER_W = TOK_PER_W // G
NCOMBO = NUM_WAVES ** G
COMBO_PAD = 16
CHUNK = 8
NCHUNK = PAIRS_PER_W // CHUNK
NBUF = 4
LANES = 16

_mesh = plsc.VectorSubcoreMesh(core_axis_name="c", subcore_axis_name="s")


@functools.partial(
    pl.kernel,
    mesh=_mesh,
    out_type=jax.ShapeDtypeStruct((N_TOKENS // G, D_G), jnp.float32),
    scratch_types=[
        pltpu.VMEM((PAIRS_PER_W,), jnp.float32),
        pltpu.VMEM((PAIRS_PER_W,), jnp.float32),
        pltpu.VMEM((PAIRS_PER_W,), jnp.float32),
        pltpu.VMEM((PAIRS_PER_W,), jnp.float32),
        pltpu.VMEM((PAIRS_PER_W,), jnp.float32),
        pltpu.VMEM((PAIRS_PER_W,), jnp.float32),
        pltpu.VMEM((PAIRS_PER_W,), jnp.int32),
        pltpu.VMEM((CHUNK, D_G), jnp.float32),
        pltpu.VMEM((CHUNK, D_G), jnp.float32),
        pltpu.VMEM((CHUNK, D_G), jnp.float32),
        pltpu.VMEM((CHUNK, D_G), jnp.float32),
        pltpu.SemaphoreType.DMA,
        pltpu.SemaphoreType.DMA,
        pltpu.SemaphoreType.DMA,
        pltpu.SemaphoreType.DMA,
        pltpu.SemaphoreType.DMA,
        pltpu.SemaphoreType.DMA,
        pltpu.SemaphoreType.DMA,
        pltpu.SemaphoreType.DMA,
    ],
)
def _sc_kernel(e0_h, e1_h, e2_h, o0_h, o1_h, o2_h, tab_h, out_h,
               e0_v, e1_v, e2_v, o0_v, o1_v, o2_v, idx_v,
               buf0, buf1, buf2, buf3,
               sg0, sg1, sg2, sg3, sw0, sw1, sw2, sw3):
    wid = lax.axis_index("s") * NUM_CORES + lax.axis_index("c")
    base = wid * PAIRS_PER_W

    pltpu.sync_copy(e0_h.at[pl.ds(base, PAIRS_PER_W)], e0_v)
    pltpu.sync_copy(e1_h.at[pl.ds(base, PAIRS_PER_W)], e1_v)
    pltpu.sync_copy(e2_h.at[pl.ds(base, PAIRS_PER_W)], e2_v)
    pltpu.sync_copy(o0_h.at[pl.ds(base, PAIRS_PER_W)], o0_v)
    pltpu.sync_copy(o1_h.at[pl.ds(base, PAIRS_PER_W)], o1_v)
    pltpu.sync_copy(o2_h.at[pl.ds(base, PAIRS_PER_W)], o2_v)

    one = jnp.full((LANES,), 1, jnp.int32)
    zero = jnp.full((LANES,), 0, jnp.int32)
    two = jnp.full((LANES,), 2, jnp.int32)
    three = jnp.full((LANES,), NUM_WAVES, jnp.int32)
    tab_off = jnp.full((LANES,), 0, jnp.int32) + wid * COMBO_PAD

    def argmax16(a0, a1, a2):
        lbl = jnp.where(a1 > a0, one, zero)
        mx = jnp.maximum(a0, a1)
        return jnp.where(a2 > mx, two, lbl)

    def label_step(i, carry):
        le = argmax16(e0_v[pl.ds(i * LANES, LANES)],
                      e1_v[pl.ds(i * LANES, LANES)],
                      e2_v[pl.ds(i * LANES, LANES)])
        lo = argmax16(o0_v[pl.ds(i * LANES, LANES)],
                      o1_v[pl.ds(i * LANES, LANES)],
                      o2_v[pl.ds(i * LANES, LANES)])
        idx_v[pl.ds(i * LANES, LANES)] = le * three + lo + tab_off
        return carry

    lax.fori_loop(0, PAIRS_PER_W // LANES, label_step, 0)

    bufs = (buf0, buf1, buf2, buf3)
    gsems = (sg0, sg1, sg2, sg3)
    wsems = (sw0, sw1, sw2, sw3)

    def start_gather(k, b):
        pltpu.async_copy(
            tab_h.at[idx_v.at[pl.ds(k * CHUNK, CHUNK)]], bufs[b], gsems[b]
        )

    def wait_gather(k, b):
        pltpu.make_async_copy(
            tab_h.at[idx_v.at[pl.ds(k * CHUNK, CHUNK)]], bufs[b], gsems[b]
        ).wait()

    def start_write(k, b):
        pltpu.async_copy(
            bufs[b], out_h.at[pl.ds(base + k * CHUNK, CHUNK)], wsems[b]
        )

    def wait_write(k, b):
        pltpu.make_async_copy(
            bufs[b], out_h.at[pl.ds(base + k * CHUNK, CHUNK)], wsems[b]
        ).wait()

    for b in range(NBUF):
        start_gather(b, b)

    NOUTER = NCHUNK // NBUF

    def chunk_step(p, carry):
        k0 = p * NBUF
        for b in range(NBUF):
            wait_gather(k0 + b, b)
            start_write(k0 + b, b)

        @pl.when(p + 1 < NOUTER)
        def _():
            for b in range(NBUF):
                wait_write(k0 + b, b)
                start_gather(k0 + NBUF + b, b)

        return carry

    lax.fori_loop(0, NOUTER, chunk_step, 0)
    for b in range(NBUF):
        wait_write(NCHUNK - NBUF + b, b)


def kernel(wave_mask, wave_embedding):
    B, S, W = wave_mask.shape
    maskP = wave_mask.reshape(B * S // G, G, W).transpose(1, 2, 0)
    pair_tab = jnp.concatenate(
        [jnp.repeat(wave_embedding, NUM_WAVES, axis=0),
         jnp.tile(wave_embedding, (NUM_WAVES, 1))], axis=1)
    tab16 = jnp.concatenate(
        [pair_tab,
         jnp.zeros((COMBO_PAD - NCOMBO, D_G), jnp.float32)], axis=0)
    tab_rep = jnp.tile(tab16, (NUM_WORKERS, 1))
    out = _sc_kernel(maskP[0, 0], maskP[0, 1], maskP[0, 2],
                     maskP[1, 0], maskP[1, 1], maskP[1, 2], tab_rep)
    return out.reshape(B, S, D_MODEL)


# speedup vs baseline: 1.6454x; 1.6454x over previous
import functools

import jax
import jax.numpy as jnp
from jax import lax
from jax.experimental import pallas as pl
from jax.experimental.pallas import tpu as pltpu
from jax.experimental.pallas import tpu_sc as plsc

D_MODEL = 1024
NUM_WAVES = 3
N_TOKENS = 4 * 8192
NUM_CORES = 2
NUM_SUBCORES = 16
NUM_WORKERS = NUM_CORES * NUM_SUBCORES

SC_TOKENS = 8192
TOK_PER_W = SC_TOKENS // NUM_WORKERS
CHUNK = 16
NCHUNK = TOK_PER_W // CHUNK
NBUF = 4
LANES = 16
TAB_PAD = 8

TC_BLK = 1024

_mesh = plsc.VectorSubcoreMesh(core_axis_name="c", subcore_axis_name="s")


@functools.partial(
    pl.kernel,
    mesh=_mesh,
    out_type=jax.ShapeDtypeStruct((SC_TOKENS, D_MODEL), jnp.float32),
    scratch_types=[
        pltpu.VMEM((TOK_PER_W,), jnp.float32),
        pltpu.VMEM((TOK_PER_W,), jnp.float32),
        pltpu.VMEM((TOK_PER_W,), jnp.float32),
        pltpu.VMEM((TOK_PER_W,), jnp.int32),
        pltpu.VMEM((CHUNK, D_MODEL), jnp.float32),
        pltpu.VMEM((CHUNK, D_MODEL), jnp.float32),
        pltpu.VMEM((CHUNK, D_MODEL), jnp.float32),
        pltpu.VMEM((CHUNK, D_MODEL), jnp.float32),
        pltpu.SemaphoreType.DMA,
        pltpu.SemaphoreType.DMA,
        pltpu.SemaphoreType.DMA,
        pltpu.SemaphoreType.DMA,
        pltpu.SemaphoreType.DMA,
        pltpu.SemaphoreType.DMA,
        pltpu.SemaphoreType.DMA,
        pltpu.SemaphoreType.DMA,
    ],
)
def _sc_kernel(m0_h, m1_h, m2_h, tab_h, out_h,
               m0_v, m1_v, m2_v, idx_v, buf0, buf1, buf2, buf3,
               sg0, sg1, sg2, sg3, sw0, sw1, sw2, sw3):
    wid = lax.axis_index("s") * NUM_CORES + lax.axis_index("c")
    base = wid * TOK_PER_W

    pltpu.sync_copy(m0_h.at[pl.ds(base, TOK_PER_W)], m0_v)
    pltpu.sync_copy(m1_h.at[pl.ds(base, TOK_PER_W)], m1_v)
    pltpu.sync_copy(m2_h.at[pl.ds(base, TOK_PER_W)], m2_v)

    one = jnp.full((LANES,), 1, jnp.int32)
    zero = jnp.full((LANES,), 0, jnp.int32)
    two = jnp.full((LANES,), 2, jnp.int32)
    tab_off = jnp.full((LANES,), 0, jnp.int32) + wid * TAB_PAD

    def label_step(i, carry):
        a0 = m0_v[pl.ds(i * LANES, LANES)]
        a1 = m1_v[pl.ds(i * LANES, LANES)]
        a2 = m2_v[pl.ds(i * LANES, LANES)]
        lbl = jnp.where(a1 > a0, one, zero)
        mx = jnp.maximum(a0, a1)
        lbl = jnp.where(a2 > mx, two, lbl)
        idx_v[pl.ds(i * LANES, LANES)] = lbl + tab_off
        return carry

    lax.fori_loop(0, TOK_PER_W // LANES, label_step, 0)

    bufs = (buf0, buf1, buf2, buf3)
    gsems = (sg0, sg1, sg2, sg3)
    wsems = (sw0, sw1, sw2, sw3)

    def start_gather(k, b):
        pltpu.async_copy(
            tab_h.at[idx_v.at[pl.ds(k * CHUNK, CHUNK)]], bufs[b], gsems[b]
        )

    def wait_gather(k, b):
        pltpu.make_async_copy(
            tab_h.at[idx_v.at[pl.ds(k * CHUNK, CHUNK)]], bufs[b], gsems[b]
        ).wait()

    def start_write(k, b):
        pltpu.async_copy(
            bufs[b], out_h.at[pl.ds(base + k * CHUNK, CHUNK)], wsems[b]
        )

    def wait_write(k, b):
        pltpu.make_async_copy(
            bufs[b], out_h.at[pl.ds(base + k * CHUNK, CHUNK)], wsems[b]
        ).wait()

    for b in range(NBUF):
        start_gather(b, b)

    NOUTER = NCHUNK // NBUF

    def chunk_step(p, carry):
        k0 = p * NBUF
        for b in range(NBUF):
            wait_gather(k0 + b, b)
            start_write(k0 + b, b)

        @pl.when(p + 1 < NOUTER)
        def _():
            for b in range(NBUF):
                wait_write(k0 + b, b)
                start_gather(k0 + NBUF + b, b)

        return carry

    lax.fori_loop(0, NOUTER, chunk_step, 0)
    for b in range(NBUF):
        wait_write(NCHUNK - NBUF + b, b)


def _tc_body(mask_ref, table_ref, out_ref):
    m = mask_ref[...]
    m0 = m[:, 0:1]
    m1 = m[:, 1:2]
    m2 = m[:, 2:3]
    l0 = jnp.logical_and(m0 >= m1, m0 >= m2)
    l1 = jnp.logical_and(jnp.logical_not(l0), m1 >= m2)
    t0 = table_ref[0:1, :]
    t1 = table_ref[1:2, :]
    t2 = table_ref[2:3, :]
    out_ref[...] = jnp.where(l0, t0, jnp.where(l1, t1, t2))


def _tc_kernel(mask2d, wave_embedding):
    n = mask2d.shape[0]
    return pl.pallas_call(
        _tc_body,
        grid=(n // TC_BLK,),
        in_specs=[
            pl.BlockSpec((TC_BLK, NUM_WAVES), lambda i: (i, 0)),
            pl.BlockSpec((NUM_WAVES, D_MODEL), lambda i: (0, 0)),
        ],
        out_specs=pl.BlockSpec((TC_BLK, D_MODEL), lambda i: (i, 0)),
        out_shape=jax.ShapeDtypeStruct((n, D_MODEL), jnp.float32),
    )(mask2d, wave_embedding)


def kernel(wave_mask, wave_embedding):
    B, S, W = wave_mask.shape
    N = B * S
    mask2d = wave_mask.reshape(N, W)
    maskT = mask2d[:SC_TOKENS].T
    tab8 = jnp.concatenate(
        [wave_embedding,
         jnp.zeros((TAB_PAD - NUM_WAVES, D_MODEL), jnp.float32)], axis=0)
    tab_rep = jnp.tile(tab8, (NUM_WORKERS, 1))
    sc_out = _sc_kernel(maskT[0], maskT[1], maskT[2], tab_rep)
    tc_out = _tc_kernel(mask2d[SC_TOKENS:], wave_embedding)
    out = jnp.concatenate([sc_out, tc_out], axis=0)
    return out.reshape(B, S, D_MODEL)


# speedup vs baseline: 4.4174x; 2.6847x over previous
import functools

import jax
import jax.numpy as jnp
from jax import lax
from jax.experimental import pallas as pl
from jax.experimental.pallas import tpu as pltpu
from jax.experimental.pallas import tpu_sc as plsc

D_MODEL = 1024
NUM_WAVES = 3
N_TOKENS = 4 * 8192
NUM_CORES = 2
NUM_SUBCORES = 16
NUM_WORKERS = NUM_CORES * NUM_SUBCORES
TOK_PER_W = N_TOKENS // NUM_WORKERS
LANES = 16
NGROUP = TOK_PER_W // LANES
LIST_ROWS = NGROUP + 1
FLAT_STRIDE = TOK_PER_W + LANES
REP = 48
NREPLICA = 8

_mesh = plsc.VectorSubcoreMesh(core_axis_name="c", subcore_axis_name="s")


@functools.partial(
    pl.kernel,
    mesh=_mesh,
    out_type=jax.ShapeDtypeStruct((N_TOKENS, D_MODEL), jnp.float32),
    scratch_types=[
        pltpu.VMEM((TOK_PER_W,), jnp.float32),
        pltpu.VMEM((TOK_PER_W,), jnp.float32),
        pltpu.VMEM((TOK_PER_W,), jnp.float32),
        pltpu.VMEM((NUM_WAVES * FLAT_STRIDE + LANES,), jnp.int32),
        pltpu.VMEM((NUM_WAVES * LIST_ROWS, LANES), jnp.int32),
        pltpu.VMEM((REP, D_MODEL), jnp.float32),
        pltpu.SemaphoreType.DMA,
    ],
)
def _sc_kernel(m0_h, m1_h, m2_h, rep_h, out_h,
               m0_v, m1_v, m2_v, flat_v, list2_v, rep_v, sem):
    wid = lax.axis_index("s") * NUM_CORES + lax.axis_index("c")
    base = wid * TOK_PER_W

    pltpu.sync_copy(
        rep_h.at[pl.ds((wid % NREPLICA) * REP, REP)], rep_v)
    pltpu.sync_copy(m0_h.at[pl.ds(base, TOK_PER_W)], m0_v)
    pltpu.sync_copy(m1_h.at[pl.ds(base, TOK_PER_W)], m1_v)
    pltpu.sync_copy(m2_h.at[pl.ds(base, TOK_PER_W)], m2_v)

    one = jnp.full((LANES,), 1, jnp.int32)
    zero = jnp.full((LANES,), 0, jnp.int32)
    two = jnp.full((LANES,), 2, jnp.int32)

    def compact_step(i, counts):
        n0, n1, n2 = counts
        a0 = m0_v[pl.ds(i * LANES, LANES)]
        a1 = m1_v[pl.ds(i * LANES, LANES)]
        a2 = m2_v[pl.ds(i * LANES, LANES)]
        lbl = jnp.where(a1 > a0, one, zero)
        mx = jnp.maximum(a0, a1)
        lbl = jnp.where(a2 > mx, two, lbl)
        gbase = base + i * LANES
        for l in range(LANES):
            lv = lbl[l]
            is0 = lv == 0
            is1 = lv == 1
            nsel = jnp.where(is0, n0, jnp.where(is1, n1, n2))
            off = lv * FLAT_STRIDE + nsel
            flat_v[pl.ds(off, LANES)] = zero + (gbase + l)
            n0 = n0 + jnp.where(is0, 1, 0)
            n1 = n1 + jnp.where(is1, 1, 0)
            n2 = n2 + jnp.where(jnp.logical_or(is0, is1), 0, 1)
        return (n0, n1, n2)

    counts = lax.fori_loop(
        0, NGROUP, compact_step,
        (jnp.int32(0), jnp.int32(0), jnp.int32(0)))

    for c in range(NUM_WAVES):
        n_c = counts[c]

        @pl.when(n_c > 0)
        def _(c=c, n_c=n_c):
            lastv = flat_v[pl.ds(c * FLAT_STRIDE + n_c - 1, LANES)]
            flat_v[pl.ds(c * FLAT_STRIDE + n_c, LANES)] = zero + lastv[0]
            nrows = (n_c + LANES - 1) // LANES

            def row_step(j, carry):
                list2_v[c * LIST_ROWS + j, :] = (
                    flat_v[pl.ds(c * FLAT_STRIDE + j * LANES, LANES)])
                return carry

            lax.fori_loop(0, nrows, row_step, 0)

            src = rep_v.at[pl.ds(c * LANES, LANES)]

            def fire_step(j, carry):
                pltpu.async_copy(
                    src, out_h.at[list2_v.at[c * LIST_ROWS + j]], sem)
                return carry

            lax.fori_loop(0, nrows, fire_step, 0)

    total_rows = sum(
        (counts[c] + LANES - 1) // LANES for c in range(NUM_WAVES))

    def drain_step(j, carry):
        pltpu.make_async_copy(
            rep_v.at[pl.ds(0, LANES)], out_h.at[list2_v.at[0]], sem).wait()
        return carry

    lax.fori_loop(0, total_rows, drain_step, 0)


def kernel(wave_mask, wave_embedding):
    B, S, W = wave_mask.shape
    maskT = wave_mask.reshape(B * S, W).T
    rep48 = jnp.repeat(wave_embedding, LANES, axis=0)
    rep_all = jnp.tile(rep48, (NREPLICA, 1))
    out = _sc_kernel(maskT[0], maskT[1], maskT[2], rep_all)
    return out.reshape(B, S, D_MODEL)
